# baseline (speedup 1.0000x reference)
import functools

import jax
import jax.numpy as jnp
from jax import lax
from jax.experimental import pallas as pl
from jax.experimental.pallas import tpu as pltpu
from jax.experimental.pallas import tpu_sc as plsc

DIM = 256
NC, NS = 2, 16
NW = NC * NS
B = 32 * 1024
B_PER_W = B // NW
CHUNK = 128
N_CHUNKS = B_PER_W // CHUNK


NBUF = 3


@functools.cache
def _build():
    mesh = plsc.VectorSubcoreMesh(core_axis_name="c", subcore_axis_name="s")

    @functools.partial(
        pl.kernel,
        mesh=mesh,
        out_type=jax.ShapeDtypeStruct((NW, B_PER_W, DIM), jnp.float32),
        scratch_types=[
            pltpu.VMEM((B_PER_W,), jnp.int32),
            pltpu.VMEM((NBUF, CHUNK, DIM), jnp.float32),
            pltpu.SemaphoreType.DMA,
            pltpu.SemaphoreType.DMA,
        ],
    )
    def gather_kernel(idx_hbm, table_hbm, out_hbm, idx_v, rows_v, gsem, ssem):
        wid = lax.axis_index("s") * NC + lax.axis_index("c")
        pltpu.sync_copy(idx_hbm.at[wid], idx_v)

        def fire_gather(j):
            return pltpu.async_copy(
                table_hbm.at[idx_v.at[pl.ds(j * CHUNK, CHUNK)]],
                rows_v.at[j % NBUF], gsem)

        def fire_scatter(j):
            return pltpu.async_copy(
                rows_v.at[j % NBUF],
                out_hbm.at[wid, pl.ds(j * CHUNK, CHUNK)],
                ssem)

        gathers = {}
        scatters = {}
        waited = set()
        for j in range(min(NBUF, N_CHUNKS)):
            gathers[j] = fire_gather(j)
        for j in range(N_CHUNKS):
            if j >= NBUF - 1 and j + 1 < N_CHUNKS:
                scatters[j + 1 - NBUF].wait()
                waited.add(j + 1 - NBUF)
                gathers[j + 1] = fire_gather(j + 1)
            gathers[j].wait()
            scatters[j] = fire_scatter(j)
        for j in range(N_CHUNKS):
            if j not in waited:
                scatters[j].wait()

    return gather_kernel


def kernel(ind, embed_weight):
    return _build()(ind, embed_weight)


# speedup vs baseline: 1.0128x; 1.0128x over previous
import functools

import jax
import jax.numpy as jnp
from jax import lax
from jax.experimental import pallas as pl
from jax.experimental.pallas import tpu as pltpu
from jax.experimental.pallas import tpu_sc as plsc

DIM = 256
NC, NS = 2, 16
NW = NC * NS
B = 32 * 1024
B_PER_W = B // NW
CHUNK = 64
N_CHUNKS = B_PER_W // CHUNK


NBUF = 7


@functools.cache
def _build():
    mesh = plsc.VectorSubcoreMesh(core_axis_name="c", subcore_axis_name="s")

    @functools.partial(
        pl.kernel,
        mesh=mesh,
        out_type=jax.ShapeDtypeStruct((NW, B_PER_W, DIM), jnp.float32),
        scratch_types=[
            pltpu.VMEM((B_PER_W,), jnp.int32),
            pltpu.VMEM((NBUF, CHUNK, DIM), jnp.float32),
            pltpu.SemaphoreType.DMA,
            pltpu.SemaphoreType.DMA,
        ],
    )
    def gather_kernel(idx_hbm, table_hbm, out_hbm, idx_v, rows_v, gsem, ssem):
        wid = lax.axis_index("s") * NC + lax.axis_index("c")
        pltpu.sync_copy(idx_hbm.at[wid], idx_v)

        def fire_gather(j):
            return pltpu.async_copy(
                table_hbm.at[idx_v.at[pl.ds(j * CHUNK, CHUNK)]],
                rows_v.at[j % NBUF], gsem)

        def fire_scatter(j):
            return pltpu.async_copy(
                rows_v.at[j % NBUF],
                out_hbm.at[wid, pl.ds(j * CHUNK, CHUNK)],
                ssem)

        gathers = {}
        scatters = {}
        waited = set()
        for j in range(min(NBUF, N_CHUNKS)):
            gathers[j] = fire_gather(j)
        for j in range(N_CHUNKS):
            if j >= NBUF - 1 and j + 1 < N_CHUNKS:
                scatters[j + 1 - NBUF].wait()
                waited.add(j + 1 - NBUF)
                gathers[j + 1] = fire_gather(j + 1)
            gathers[j].wait()
            scatters[j] = fire_scatter(j)
        for j in range(N_CHUNKS):
            if j not in waited:
                scatters[j].wait()

    return gather_kernel


def kernel(ind, embed_weight):
    return _build()(ind.astype(jnp.int32), embed_weight)
